# baseline (speedup 1.0000x reference)
import functools

import jax
import jax.numpy as jnp
from jax import lax
from jax.experimental import pallas as pl
from jax.experimental.pallas import tpu as pltpu
from jax.experimental.pallas import tpu_sc as plsc

N = 100000
M = 64
K = 32
G = 16
NGROUPS = N // G
BIN = G * M
BOUT = G * K
NC = 2
NS = 16
NW = NC * NS
NBUF = 2
T = (NGROUPS + NW - 1) // NW
assert T % NBUF == 0


def _batcher(n):
    comps = []

    def merge(lo, n_, r):
        m = r * 2
        if m < n_:
            merge(lo, n_, m)
            merge(lo + r, n_, m)
            for i in range(lo + r, lo + n_ - r, m):
                comps.append((i, i + r))
        else:
            comps.append((lo, lo + r))

    def sort(lo, n_):
        if n_ > 1:
            m = n_ // 2
            sort(lo, m)
            sort(lo + m, m)
            merge(lo, n_, 1)

    sort(0, n)
    return comps


def _network():
    comps = [c for c in _batcher(M) if c[0] != 0]
    needed = set(range(1, K))
    kept = []
    for (i, j) in reversed(comps):
        if i in needed or j in needed:
            kept.append((i, j))
            needed.add(i)
            needed.add(j)
    kept.reverse()
    return kept


_COMPS = _network()


def _sc_sort(dist_hbm, nidx_hbm, outd_hbm, outn_hbm, *refs):
    dist_v = refs[0:NBUF]
    nidx_v = refs[NBUF:2 * NBUF]
    outd_v = refs[2 * NBUF:3 * NBUF]
    outn_v = refs[3 * NBUF:4 * NBUF]
    ind_s = refs[4 * NBUF:5 * NBUF]
    inn_s = refs[5 * NBUF:6 * NBUF]
    outd_s = refs[6 * NBUF:7 * NBUF]
    outn_s = refs[7 * NBUF:8 * NBUF]

    wid = lax.axis_index("s") * NC + lax.axis_index("c")
    rows = lax.iota(jnp.int32, G)

    def fetch(t, b):
        base = (wid + NW * t) * BIN
        pltpu.make_async_copy(
            dist_hbm.at[pl.ds(base, BIN)], dist_v[b], ind_s[b]).start()
        pltpu.make_async_copy(
            nidx_hbm.at[pl.ds(base, BIN)], nidx_v[b], inn_s[b]).start()

    for b in range(NBUF):
        fetch(b, b)

    def body(tt, carry):
        for b in range(NBUF):
            t = NBUF * tt + b
            g = wid + NW * t

            @pl.when(g < NGROUPS)
            def _():
                in_base = g * BIN
                out_base = g * BOUT
                pltpu.make_async_copy(
                    dist_hbm.at[pl.ds(in_base, BIN)], dist_v[b],
                    ind_s[b]).wait()
                pltpu.make_async_copy(
                    nidx_hbm.at[pl.ds(in_base, BIN)], nidx_v[b],
                    inn_s[b]).wait()

                keys = [None] * M
                cols = [None] * M
                for j in range(1, M):
                    keys[j] = dist_v[b][pl.ds(j * G, G)]
                    cols[j] = jnp.full((G,), j, jnp.int32)

                for (i, j) in _COMPS:
                    ka, kb = keys[i], keys[j]
                    pa, pb = cols[i], cols[j]
                    swap = kb < ka
                    keys[i] = jnp.minimum(ka, kb)
                    keys[j] = jnp.maximum(ka, kb)
                    cols[i] = jnp.where(swap, pb, pa)
                    cols[j] = jnp.where(swap, pa, pb)

                @pl.when(t >= NBUF)
                def _():
                    pltpu.make_async_copy(
                        outd_v[b], outd_hbm.at[pl.ds(out_base, BOUT)],
                        outd_s[b]).wait()
                    pltpu.make_async_copy(
                        outn_v[b], outn_hbm.at[pl.ds(out_base, BOUT)],
                        outn_s[b]).wait()

                outd_v[b][pl.ds(0, G)] = dist_v[b][pl.ds(0, G)]
                outn_v[b][pl.ds(0, G)] = nidx_v[b][pl.ds(0, G)]
                for p in range(1, K):
                    outd_v[b][pl.ds(p * G, G)] = keys[p]
                    outn_v[b][pl.ds(p * G, G)] = plsc.load_gather(
                        nidx_v[b], [(cols[p] * G) + rows])

                pltpu.make_async_copy(
                    outd_v[b], outd_hbm.at[pl.ds(out_base, BOUT)],
                    outd_s[b]).start()
                pltpu.make_async_copy(
                    outn_v[b], outn_hbm.at[pl.ds(out_base, BOUT)],
                    outn_s[b]).start()

                @pl.when(g + NBUF * NW < NGROUPS)
                def _():
                    fetch(t + NBUF, b)

        return carry

    lax.fori_loop(0, T // NBUF, body, 0)

    for b in range(NBUF):
        @pl.when(wid + NW * (T - NBUF + b) < NGROUPS)
        def _():
            pltpu.make_async_copy(
                outd_v[b], outd_hbm.at[pl.ds(0, BOUT)], outd_s[b]).wait()
            pltpu.make_async_copy(
                outn_v[b], outn_hbm.at[pl.ds(0, BOUT)], outn_s[b]).wait()


@jax.jit
def kernel(distances, nidx):
    run = functools.partial(
        pl.kernel,
        out_type=(jax.ShapeDtypeStruct((N * K,), jnp.float32),
                  jax.ShapeDtypeStruct((N * K,), jnp.int32)),
        mesh=plsc.VectorSubcoreMesh(core_axis_name="c", subcore_axis_name="s"),
        compiler_params=pltpu.CompilerParams(needs_layout_passes=False),
        scratch_types=(
            [pltpu.VMEM((BIN,), jnp.float32)] * NBUF
            + [pltpu.VMEM((BIN,), jnp.int32)] * NBUF
            + [pltpu.VMEM((BOUT,), jnp.float32)] * NBUF
            + [pltpu.VMEM((BOUT,), jnp.int32)] * NBUF
            + [pltpu.SemaphoreType.DMA] * (4 * NBUF)
        ),
    )(_sc_sort)
    dt = distances.reshape(NGROUPS, G, M).transpose(0, 2, 1).reshape(-1)
    nt = nidx.reshape(NGROUPS, G, M).transpose(0, 2, 1).reshape(-1)
    sdist, snidx = run(dt, nt)
    sdist = sdist.reshape(NGROUPS, K, G).transpose(0, 2, 1).reshape(N, K)
    snidx = snidx.reshape(NGROUPS, K, G).transpose(0, 2, 1).reshape(N, K)
    return sdist, snidx


# speedup vs baseline: 1.3689x; 1.3689x over previous
import functools

import jax
import jax.numpy as jnp
from jax import lax
from jax.experimental import pallas as pl
from jax.experimental.pallas import tpu as pltpu
from jax.experimental.pallas import tpu_sc as plsc

N = 100000
M = 64
K = 32
G = 16
NGROUPS = N // G
NC = 2
NS = 16
NW = NC * NS
NBUF = 2
T = (NGROUPS + NW - 1) // NW
assert T % NBUF == 0


def _batcher(n):
    comps = []

    def merge(lo, n_, r):
        m = r * 2
        if m < n_:
            merge(lo, n_, m)
            merge(lo + r, n_, m)
            for i in range(lo + r, lo + n_ - r, m):
                comps.append((i, i + r))
        else:
            comps.append((lo, lo + r))

    def sort(lo, n_):
        if n_ > 1:
            m = n_ // 2
            sort(lo, m)
            sort(lo + m, m)
            merge(lo, n_, 1)

    sort(0, n)
    return comps


def _network():
    comps = [c for c in _batcher(M) if c[0] != 0]
    needed = set(range(1, K))
    kept = []
    for (i, j) in reversed(comps):
        if i in needed or j in needed:
            kept.append((i, j))
            needed.add(i)
            needed.add(j)
    kept.reverse()
    return kept


_COMPS = _network()


def _sc_sort(dist_hbm, nidx_hbm, outd_hbm, outn_hbm, *refs):
    dist_v = refs[0:NBUF]
    nidx_v = refs[NBUF:2 * NBUF]
    outd_v = refs[2 * NBUF:3 * NBUF]
    outn_v = refs[3 * NBUF:4 * NBUF]
    ind_s = refs[4 * NBUF:5 * NBUF]
    inn_s = refs[5 * NBUF:6 * NBUF]
    outd_s = refs[6 * NBUF:7 * NBUF]
    outn_s = refs[7 * NBUF:8 * NBUF]
    rotd_v = refs[8 * NBUF]

    wid = lax.axis_index("s") * NC + lax.axis_index("c")
    rows = lax.iota(jnp.int32, G)

    def fetch(t, b):
        base = (wid + NW * t) * G
        pltpu.make_async_copy(
            dist_hbm.at[pl.ds(base, G)],
            dist_v[b], ind_s[b]).start()
        pltpu.make_async_copy(
            nidx_hbm.at[pl.ds(base, G)],
            nidx_v[b], inn_s[b]).start()

    for b in range(NBUF):
        fetch(b, b)

    def body(tt, carry):
        for b in range(NBUF):
            t = NBUF * tt + b
            g = wid + NW * t

            @pl.when(g < NGROUPS)
            def _():
                base = g * G
                pltpu.make_async_copy(
                    dist_hbm.at[pl.ds(base, G)],
                    dist_v[b], ind_s[b]).wait()
                pltpu.make_async_copy(
                    nidx_hbm.at[pl.ds(base, G)],
                    nidx_v[b], inn_s[b]).wait()

                for r in range(G):
                    rsp = jnp.full((G,), r, jnp.int32)
                    for q in range(M // G):
                        v = dist_v[b][r, pl.ds(q * G, G)]
                        cidx = (rows + (q * G + r)) & (M - 1)
                        plsc.store_scatter(rotd_v, [rsp, cidx], v)

                keys = [None] * M
                cols = [None] * M
                for j in range(1, M):
                    keys[j] = plsc.load_gather(rotd_v, [rows, (rows + j) & (M - 1)])
                    cols[j] = jnp.full((G,), j, jnp.int32)

                for (i, j) in _COMPS:
                    ka, kb = keys[i], keys[j]
                    pa, pb = cols[i], cols[j]
                    swap = kb < ka
                    keys[i] = jnp.minimum(ka, kb)
                    keys[j] = jnp.maximum(ka, kb)
                    cols[i] = jnp.where(swap, pb, pa)
                    cols[j] = jnp.where(swap, pa, pb)

                @pl.when(t >= NBUF)
                def _():
                    pltpu.make_async_copy(
                        outd_v[b],
                        outd_hbm.at[pl.ds(base, G)], outd_s[b]).wait()
                    pltpu.make_async_copy(
                        outn_v[b],
                        outn_hbm.at[pl.ds(base, G)], outn_s[b]).wait()

                c0 = jnp.full((G,), 0, jnp.int32)
                plsc.store_scatter(outd_v[b], [rows, c0],
                                   plsc.load_gather(rotd_v, [rows, rows]))
                plsc.store_scatter(outn_v[b], [rows, c0],
                                   plsc.load_gather(nidx_v[b], [rows, c0]))
                for p in range(1, K):
                    cp = jnp.full((G,), p, jnp.int32)
                    plsc.store_scatter(outd_v[b], [rows, cp], keys[p])
                    plsc.store_scatter(outn_v[b], [rows, cp],
                                       plsc.load_gather(nidx_v[b],
                                                        [rows, cols[p]]))

                pltpu.make_async_copy(
                    outd_v[b],
                    outd_hbm.at[pl.ds(base, G)], outd_s[b]).start()
                pltpu.make_async_copy(
                    outn_v[b],
                    outn_hbm.at[pl.ds(base, G)], outn_s[b]).start()

                @pl.when(g + NBUF * NW < NGROUPS)
                def _():
                    fetch(t + NBUF, b)

        return carry

    lax.fori_loop(0, T // NBUF, body, 0)

    for b in range(NBUF):
        @pl.when(wid + NW * (T - NBUF + b) < NGROUPS)
        def _():
            pltpu.make_async_copy(
                outd_v[b],
                outd_hbm.at[pl.ds(0, G)], outd_s[b]).wait()
            pltpu.make_async_copy(
                outn_v[b],
                outn_hbm.at[pl.ds(0, G)], outn_s[b]).wait()


@jax.jit
def kernel(distances, nidx):
    run = functools.partial(
        pl.kernel,
        out_type=(jax.ShapeDtypeStruct((N, K), jnp.float32),
                  jax.ShapeDtypeStruct((N, K), jnp.int32)),
        mesh=plsc.VectorSubcoreMesh(core_axis_name="c", subcore_axis_name="s"),
        compiler_params=pltpu.CompilerParams(needs_layout_passes=False),
        scratch_types=(
            [pltpu.VMEM((G, M), jnp.float32)] * NBUF
            + [pltpu.VMEM((G, M), jnp.int32)] * NBUF
            + [pltpu.VMEM((G, K), jnp.float32)] * NBUF
            + [pltpu.VMEM((G, K), jnp.int32)] * NBUF
            + [pltpu.SemaphoreType.DMA] * (4 * NBUF)
            + [pltpu.VMEM((G, M), jnp.float32)]
        ),
    )(_sc_sort)
    return run(distances, nidx)
